# baseline (speedup 1.0000x reference)
import functools

import jax
import jax.numpy as jnp
from jax import lax
from jax.experimental import pallas as pl
from jax.experimental.pallas import tpu as pltpu
from jax.experimental.pallas import tpu_sc as plsc

V, D, B = 1000000, 64, 16384
NC, NS = 2, 16
NW = NC * NS
B_PER_W = B // NW

_mesh = plsc.VectorSubcoreMesh(core_axis_name="c", subcore_axis_name="s")


@functools.partial(
    pl.kernel,
    mesh=_mesh,
    out_type=jax.ShapeDtypeStruct((D, B), jnp.float32),
    scratch_types=[
        pltpu.VMEM((B_PER_W,), jnp.int32),
        pltpu.VMEM((D, B_PER_W), jnp.float32),
        pltpu.SemaphoreType.DMA,
    ],
    compiler_params=pltpu.CompilerParams(use_tc_tiling_on_sc=False),
)
def _gather_sc(table_t, idx_hbm, out_t, idx_v, cols_v, sem):
    wid = lax.axis_index("s") * NC + lax.axis_index("c")
    base = wid * B_PER_W
    pltpu.sync_copy(idx_hbm.at[wid], idx_v)
    copies = [
        pltpu.async_copy(table_t.at[c].at[idx_v], cols_v.at[c], sem)
        for c in range(D)
    ]
    for cp in copies:
        cp.wait()
    pltpu.sync_copy(cols_v, out_t.at[:, pl.ds(base, B_PER_W)])


def kernel(input, indices):
    idx = indices.astype(jnp.int32).reshape(NW, B_PER_W)
    out_t = _gather_sc(input.T, idx)
    return out_t.T


# speedup vs baseline: 7.8839x; 7.8839x over previous
import functools

import jax
import jax.numpy as jnp
from jax import lax
from jax.experimental import pallas as pl
from jax.experimental.pallas import tpu as pltpu
from jax.experimental.pallas import tpu_sc as plsc

V, D, B = 1000000, 64, 16384
NC, NS = 2, 16
NW = NC * NS
B_PER_W = B // NW
CH = 128
NCH = B_PER_W // CH
L = 16

_mesh = plsc.VectorSubcoreMesh(core_axis_name="c", subcore_axis_name="s")


@functools.partial(
    pl.kernel,
    mesh=_mesh,
    out_type=jax.ShapeDtypeStruct((B, D), jnp.float32),
    scratch_types=[
        pltpu.VMEM((NCH, CH), jnp.int32),
        pltpu.VMEM((B_PER_W,), jnp.int32),
        pltpu.VMEM((CH, 2 * D), jnp.float32),
        pltpu.VMEM((CH, D), jnp.float32),
        pltpu.SemaphoreType.DMA,
    ],
)
def _gather_sc(table2, pidx_hbm, hoff_hbm, out_hbm,
               pidx_v, hoff_v, rows_v, out_v, sem):
    wid = lax.axis_index("s") * NC + lax.axis_index("c")
    base = wid * B_PER_W
    pltpu.sync_copy(pidx_hbm.at[wid], pidx_v)
    pltpu.sync_copy(hoff_hbm.at[wid], hoff_v)
    for j in range(NCH):
        pltpu.async_copy(table2.at[pidx_v.at[j]], rows_v, sem).wait()
        for g in range(CH // L):
            h_vec = hoff_v[pl.ds(j * CH + g * L, L)]
            for l in range(L):
                i = g * L + l
                h = h_vec[l]
                for c in range(D // L):
                    out_v[i, pl.ds(c * L, L)] = rows_v[i, pl.ds(h + c * L, L)]
        pltpu.sync_copy(out_v, out_hbm.at[pl.ds(base + j * CH, CH)])


def kernel(input, indices):
    idx = indices.astype(jnp.int32)
    table2 = input.reshape(V // 2, 2 * D)
    pidx = (idx >> 1).reshape(NW, NCH, CH)
    hoff = ((idx & 1) * D).reshape(NW, B_PER_W)
    return _gather_sc(table2, pidx, hoff)


# speedup vs baseline: 8.9282x; 1.1325x over previous
import functools

import jax
import jax.numpy as jnp
from jax import lax
from jax.experimental import pallas as pl
from jax.experimental.pallas import tpu as pltpu
from jax.experimental.pallas import tpu_sc as plsc

V, D, B = 1000000, 64, 16384
NC, NS = 2, 16
NW = NC * NS
B_PER_W = B // NW
CH = 128
NCH = B_PER_W // CH

_mesh = plsc.VectorSubcoreMesh(core_axis_name="c", subcore_axis_name="s")


@functools.partial(
    pl.kernel,
    mesh=_mesh,
    out_type=jax.ShapeDtypeStruct((B, D), jnp.float32),
    scratch_types=[
        pltpu.VMEM((NCH, CH), jnp.int32),
        pltpu.VMEM((CH, 2 * D), jnp.float32),
        pltpu.VMEM((CH, D), jnp.float32),
        pltpu.SemaphoreType.DMA,
    ],
)
def _gather_sc(tablep, idx_hbm, out_hbm, idx_v, rows_v, out_v, sem):
    wid = lax.axis_index("s") * NC + lax.axis_index("c")
    base = wid * B_PER_W
    pltpu.sync_copy(idx_hbm.at[wid], idx_v)
    L = 16
    for j in range(NCH):
        pltpu.async_copy(tablep.at[idx_v.at[j]], rows_v, sem).wait()

        @pl.loop(0, CH)
        def _(i):
            for c in range(D // L):
                out_v[i, pl.ds(c * L, L)] = rows_v[i, pl.ds(c * L, L)]

        pltpu.sync_copy(out_v, out_hbm.at[pl.ds(base + j * CH, CH)])


def kernel(input, indices):
    idx = indices.astype(jnp.int32).reshape(NW, NCH, CH)
    tablep = jnp.pad(input, ((0, 0), (0, D)))
    return _gather_sc(tablep, idx)
